# Initial kernel scaffold; baseline (speedup 1.0000x reference)
import functools

import jax
import jax.numpy as jnp
from jax import lax
from jax.experimental import pallas as pl
from jax.experimental.pallas import tpu as pltpu
from jax.experimental.pallas import tpu_sc as plsc

N = 10000
E = 320000
D_IN = 64
C = 128
B = 1024
V = 100000
OUT = 1

NC = 2
NS = 16
NW = NC * NS

CHUNK = 128
NCHUNKS = E // CHUNK
ROWS_PER_TILE = N // NS
CNT_W = 16

_mesh = plsc.VectorSubcoreMesh(core_axis_name="c", subcore_axis_name="s")


@functools.partial(
    pl.kernel,
    out_type=(
        jax.ShapeDtypeStruct((N, C), jnp.float32),
        jax.ShapeDtypeStruct((N,), jnp.float32),
    ),
    mesh=_mesh,
    scratch_types=[
        pltpu.VMEM((CHUNK,), jnp.int32),
        pltpu.VMEM((CHUNK, C), jnp.float32),
        pltpu.VMEM((B,), jnp.int32),
        pltpu.VMEM((CHUNK,), jnp.int32),
        pltpu.VMEM((CHUNK,), jnp.int32),
        pltpu.VMEM((CHUNK,), jnp.float32),
        pltpu.SemaphoreType.DMA,
    ],
)
def _sc_prep(emb_hbm, nid_hbm, seed_hbm, batch_hbm, time_hbm,
             erows_hbm, rel_hbm,
             nid_v, erow_v, seed_v, batch_v, time_v, rel_v, sem):
    wid = lax.axis_index("s") * NC + lax.axis_index("c")
    pltpu.sync_copy(seed_hbm, seed_v)
    nck = jnp.where(wid < 14, 3, 2)
    base = jnp.where(wid < 14, 3 * wid, 2 * wid + 14)

    def chunk_body(j, carry):
        off = (base + j) * CHUNK
        pltpu.sync_copy(nid_hbm.at[pl.ds(off, CHUNK)], nid_v)
        pltpu.async_copy(emb_hbm.at[nid_v], erow_v, sem).wait()
        pltpu.sync_copy(erow_v, erows_hbm.at[pl.ds(off, CHUNK)])
        pltpu.sync_copy(batch_hbm.at[pl.ds(off, CHUNK)], batch_v)
        pltpu.sync_copy(time_hbm.at[pl.ds(off, CHUNK)], time_v)
        for i in range(CHUNK // 16):
            idx = batch_v[pl.ds(i * 16, 16)]
            st = plsc.load_gather(seed_v, [idx])
            tt = time_v[pl.ds(i * 16, 16)]
            rel_v[pl.ds(i * 16, 16)] = (st - tt).astype(jnp.float32) / 86400.0
        pltpu.sync_copy(rel_v, rel_hbm.at[pl.ds(off, CHUNK)])
        return carry

    lax.fori_loop(0, nck, chunk_body, 0)

    @pl.when(wid == NW - 1)
    def _tail():
        toff = N - 16
        pltpu.sync_copy(nid_hbm.at[pl.ds(toff, 16)], nid_v.at[pl.ds(0, 16)])
        pltpu.async_copy(emb_hbm.at[nid_v.at[pl.ds(0, 16)]],
                         erow_v.at[pl.ds(0, 16)], sem).wait()
        pltpu.sync_copy(erow_v.at[pl.ds(0, 16)],
                        erows_hbm.at[pl.ds(toff, 16)])
        pltpu.sync_copy(batch_hbm.at[pl.ds(toff, 16)], batch_v.at[pl.ds(0, 16)])
        pltpu.sync_copy(time_hbm.at[pl.ds(toff, 16)], time_v.at[pl.ds(0, 16)])
        idx = batch_v[pl.ds(0, 16)]
        st = plsc.load_gather(seed_v, [idx])
        tt = time_v[pl.ds(0, 16)]
        rel_v[pl.ds(0, 16)] = (st - tt).astype(jnp.float32) / 86400.0
        pltpu.sync_copy(rel_v.at[pl.ds(0, 16)], rel_hbm.at[pl.ds(toff, 16)])


def _make_sc_agg(with_cnt: bool):
    out_type = [jax.ShapeDtypeStruct((NC, N, C), jnp.float32)]
    scratch = [
        pltpu.VMEM((CHUNK,), jnp.int32),
        pltpu.VMEM((1, CHUNK), jnp.int32),
        pltpu.VMEM((CHUNK, C), jnp.float32),
        pltpu.VMEM((125, C), jnp.float32),
        pltpu.VMEM_SHARED((N, C), jnp.float32),
        pltpu.SemaphoreType.DMA,
    ]
    if with_cnt:
        out_type.append(jax.ShapeDtypeStruct((NC, N, CNT_W), jnp.float32))
        scratch += [
            pltpu.VMEM((CHUNK, CNT_W), jnp.float32),
            pltpu.VMEM((ROWS_PER_TILE, CNT_W), jnp.float32),
            pltpu.VMEM_SHARED((N, CNT_W), jnp.float32),
        ]

    def body(h_hbm, src_hbm, dst_hbm, z128_hbm, *rest):
        if with_cnt:
            (ones_hbm, p_hbm, cnt_hbm, src_v, dst_v, rows_v, zb_v, agg_sh,
             sem, ones_v, cb_v, cnt_sh) = rest
        else:
            (p_hbm, src_v, dst_v, rows_v, zb_v, agg_sh, sem) = rest
        cid = lax.axis_index("c")
        sid = lax.axis_index("s")
        wid = sid * NC + cid
        row0 = sid * ROWS_PER_TILE

        pltpu.sync_copy(z128_hbm, zb_v)
        for r in range(5):
            pltpu.sync_copy(zb_v, agg_sh.at[pl.ds(row0 + r * 125, 125)])
        if with_cnt:
            pltpu.sync_copy(ones_hbm, ones_v)
            pltpu.sync_copy(z128_hbm.at[pl.ds(0, ROWS_PER_TILE // 8),
                                        pl.ds(0, CNT_W)], cb_v)
            pltpu.sync_copy(cb_v, cnt_sh.at[pl.ds(row0, ROWS_PER_TILE)])
        plsc.subcore_barrier()

        nck = jnp.where(wid < 4, 79, 78)
        base = wid * 78 + jnp.minimum(wid, 4)

        def chunk_body(j, carry):
            off = (base + j) * CHUNK
            pltpu.sync_copy(src_hbm.at[pl.ds(off, CHUNK)], src_v)
            pltpu.sync_copy(dst_hbm.at[pl.ds(off, CHUNK)], dst_v.at[0])
            pltpu.async_copy(h_hbm.at[src_v], rows_v, sem).wait()
            pltpu.sync_copy(rows_v, agg_sh.at[dst_v.at[0]], add=True)
            if with_cnt:
                pltpu.sync_copy(ones_v, cnt_sh.at[dst_v.at[0]], add=True)
            return carry

        lax.fori_loop(0, nck, chunk_body, 0)
        plsc.subcore_barrier()

        for r in range(5):
            pltpu.sync_copy(agg_sh.at[pl.ds(row0 + r * 125, 125)], zb_v)
            pltpu.sync_copy(zb_v, p_hbm.at[cid, pl.ds(row0 + r * 125, 125)])
        if with_cnt:
            pltpu.sync_copy(cnt_sh.at[pl.ds(row0, ROWS_PER_TILE)], cb_v)
            pltpu.sync_copy(cb_v, cnt_hbm.at[cid, pl.ds(row0, ROWS_PER_TILE)])

    return pl.kernel(body, out_type=tuple(out_type) if with_cnt else out_type[0],
                     mesh=_mesh, scratch_types=scratch)


_sc_agg_cnt = _make_sc_agg(True)
_sc_agg = _make_sc_agg(False)


_BLK = 1000


def _enc_body(x_ref, rel_ref, er_ref, we_ref, wt_ref, be_ref, bt_ref, o_ref):
    rel = rel_ref[...]
    j = lax.broadcasted_iota(jnp.float32, (1, C // 2), 1)
    w = jnp.exp(j * (-2.0 * jnp.log(10000.0) / C))
    ang = rel * w
    pe = jnp.concatenate([jnp.sin(ang), jnp.cos(ang)], axis=1)
    h = jnp.dot(x_ref[...], we_ref[...], preferred_element_type=jnp.float32)
    h = h + jnp.dot(pe, wt_ref[...], preferred_element_type=jnp.float32)
    o_ref[...] = h + be_ref[...] + bt_ref[...] + er_ref[...]


def _tc_encode(x, rel, erows, W_enc, W_time, b_enc, b_time):
    return pl.pallas_call(
        _enc_body,
        grid=(N // _BLK,),
        in_specs=[
            pl.BlockSpec((_BLK, D_IN), lambda i: (i, 0)),
            pl.BlockSpec((_BLK, 1), lambda i: (i, 0)),
            pl.BlockSpec((_BLK, C), lambda i: (i, 0)),
            pl.BlockSpec((D_IN, C), lambda i: (0, 0)),
            pl.BlockSpec((C, C), lambda i: (0, 0)),
            pl.BlockSpec((1, C), lambda i: (0, 0)),
            pl.BlockSpec((1, C), lambda i: (0, 0)),
        ],
        out_specs=pl.BlockSpec((_BLK, C), lambda i: (i, 0)),
        out_shape=jax.ShapeDtypeStruct((N, C), jnp.float32),
    )(x, rel, erows, W_enc, W_time, b_enc, b_time)


def _comb_body(h_ref, p0_ref, p1_ref, c0_ref, c1_ref, ws_ref, wn_ref, b_ref,
               o_ref):
    cnt = c0_ref[...][:, :1] + c1_ref[...][:, :1]
    agg = (p0_ref[...] + p1_ref[...]) / jnp.maximum(cnt, 1.0)
    h = jnp.dot(h_ref[...], ws_ref[...], preferred_element_type=jnp.float32)
    h = h + jnp.dot(agg, wn_ref[...], preferred_element_type=jnp.float32)
    o_ref[...] = jnp.maximum(h + b_ref[...], 0.0)


def _tc_combine(h, p0, p1, c0, c1, Ws, Wn, b):
    return pl.pallas_call(
        _comb_body,
        grid=(N // _BLK,),
        in_specs=[
            pl.BlockSpec((_BLK, C), lambda i: (i, 0)),
            pl.BlockSpec((_BLK, C), lambda i: (i, 0)),
            pl.BlockSpec((_BLK, C), lambda i: (i, 0)),
            pl.BlockSpec((_BLK, CNT_W), lambda i: (i, 0)),
            pl.BlockSpec((_BLK, CNT_W), lambda i: (i, 0)),
            pl.BlockSpec((C, C), lambda i: (0, 0)),
            pl.BlockSpec((C, C), lambda i: (0, 0)),
            pl.BlockSpec((1, C), lambda i: (0, 0)),
        ],
        out_specs=pl.BlockSpec((_BLK, C), lambda i: (i, 0)),
        out_shape=jax.ShapeDtypeStruct((N, C), jnp.float32),
    )(h, p0, p1, c0, c1, Ws, Wn, b)


def _final_body(h_ref, p0_ref, p1_ref, c0_ref, c1_ref, ws_ref, wn_ref, b_ref,
                wh_ref, bh_ref, o_ref):
    cnt = c0_ref[...][:, :1] + c1_ref[...][:, :1]
    agg = (p0_ref[...] + p1_ref[...]) / jnp.maximum(cnt, 1.0)
    h = jnp.dot(h_ref[...], ws_ref[...], preferred_element_type=jnp.float32)
    h = h + jnp.dot(agg, wn_ref[...], preferred_element_type=jnp.float32)
    h = jnp.maximum(h + b_ref[...], 0.0)
    o_ref[...] = jnp.dot(h, wh_ref[...],
                         preferred_element_type=jnp.float32) + bh_ref[...]


def _tc_final(h, p0, p1, c0, c1, Ws, Wn, b, W_head, b_head):
    return pl.pallas_call(
        _final_body,
        grid=(1,),
        in_specs=[
            pl.BlockSpec((B, C), lambda i: (0, 0)),
            pl.BlockSpec((B, C), lambda i: (0, 0)),
            pl.BlockSpec((B, C), lambda i: (0, 0)),
            pl.BlockSpec((B, CNT_W), lambda i: (0, 0)),
            pl.BlockSpec((B, CNT_W), lambda i: (0, 0)),
            pl.BlockSpec((C, C), lambda i: (0, 0)),
            pl.BlockSpec((C, C), lambda i: (0, 0)),
            pl.BlockSpec((1, C), lambda i: (0, 0)),
            pl.BlockSpec((C, OUT), lambda i: (0, 0)),
            pl.BlockSpec((1, OUT), lambda i: (0, 0)),
        ],
        out_specs=pl.BlockSpec((B, OUT), lambda i: (0, 0)),
        out_shape=jax.ShapeDtypeStruct((B, OUT), jnp.float32),
    )(h, p0, p1, c0, c1, Ws, Wn, b, W_head, b_head)


def kernel(x, edge_index, seed_time, time, batch, n_id,
           W_enc, b_enc, W_time, b_time, emb,
           W_self0, W_neigh0, b0, W_self1, W_neigh1, b1,
           W_head, b_head):
    src = edge_index[0]
    dst = edge_index[1]
    z128 = jnp.zeros((125, C), jnp.float32)
    ones16 = jnp.ones((CHUNK, CNT_W), jnp.float32)

    erows, rel = _sc_prep(emb, n_id, seed_time, batch, time)
    h0 = _tc_encode(x, rel.reshape(N, 1), erows, W_enc, W_time,
                    b_enc.reshape(1, C), b_time.reshape(1, C))

    p, cnt = _sc_agg_cnt(h0, src, dst, z128, ones16)
    h1 = _tc_combine(h0, p[0], p[1], cnt[0], cnt[1], W_self0, W_neigh0,
                     b0.reshape(1, C))

    p2 = _sc_agg(h1, src, dst, z128)
    out = _tc_final(h1, p2[0], p2[1], cnt[0], cnt[1], W_self1, W_neigh1,
                    b1.reshape(1, C), W_head, b_head.reshape(1, OUT))
    return out


# speedup vs baseline: 5.3534x; 5.3534x over previous
import functools

import jax
import jax.numpy as jnp
from jax import lax
from jax.experimental import pallas as pl
from jax.experimental.pallas import tpu as pltpu
from jax.experimental.pallas import tpu_sc as plsc

N = 10000
E = 320000
D_IN = 64
C = 128
B = 1024
V = 100000
OUT = 1

NC = 2
NS = 16
NW = NC * NS

CHUNK = 128
NCHUNKS = E // CHUNK
RPT = 624

_mesh = plsc.VectorSubcoreMesh(core_axis_name="c", subcore_axis_name="s",
                               num_cores=NC, num_subcores=NS)


@functools.partial(
    pl.kernel,
    out_type=jax.ShapeDtypeStruct((N, C), jnp.float32),
    mesh=_mesh,
    scratch_types=[
        pltpu.VMEM((CHUNK,), jnp.int32),
        pltpu.VMEM((CHUNK, C), jnp.float32),
        pltpu.SemaphoreType.DMA,
    ],
)
def _sc_prep(emb_hbm, nid_hbm, erows_hbm, nid_v, erow_v, sem):
    wid = lax.axis_index("s") * NC + lax.axis_index("c")
    nck = jnp.where(wid < 14, 3, 2)
    base = jnp.where(wid < 14, 3 * wid, 2 * wid + 14)

    def chunk_body(j, carry):
        off = (base + j) * CHUNK
        pltpu.sync_copy(nid_hbm.at[pl.ds(off, CHUNK)], nid_v)
        pltpu.async_copy(emb_hbm.at[nid_v], erow_v, sem).wait()
        pltpu.sync_copy(erow_v, erows_hbm.at[pl.ds(off, CHUNK)])
        return carry

    lax.fori_loop(0, nck, chunk_body, 0)

    @pl.when(wid == NW - 1)
    def _tail():
        toff = N - 16
        pltpu.sync_copy(nid_hbm.at[pl.ds(toff, 16)], nid_v.at[pl.ds(0, 16)])
        pltpu.async_copy(emb_hbm.at[nid_v.at[pl.ds(0, 16)]],
                         erow_v.at[pl.ds(0, 16)], sem).wait()
        pltpu.sync_copy(erow_v.at[pl.ds(0, 16)],
                        erows_hbm.at[pl.ds(toff, 16)])


_PIECES = (0, 128, 256, 384, 496)


def _edge_range(wid):
    nck = jnp.where(wid < 4, 79, 78)
    base = wid * 78 + jnp.minimum(wid, 4)
    return nck, base


def _zero_acc(acc_sh, rowidx_hbm, z_hbm, rows_v, src_v, idx16_v, sid):
    pltpu.sync_copy(z_hbm, rows_v)
    row0 = sid * RPT
    for r in _PIECES:
        pltpu.sync_copy(rowidx_hbm.at[pl.ds(row0 + r, CHUNK)], src_v)
        pltpu.sync_copy(rows_v, acc_sh.at[src_v])

    @pl.when(sid == NS - 1)
    def _zero_tail():
        pltpu.sync_copy(rowidx_hbm.at[pl.ds(N - 16, 16)], idx16_v)
        pltpu.sync_copy(rows_v.at[pl.ds(0, 16)], acc_sh.at[idx16_v])


def _copy_out(acc_sh, rowidx_hbm, p_hbm, rows_v, src_v, idx16_v, sid, cid,
              sem):
    row0 = sid * RPT
    for r in _PIECES:
        pltpu.sync_copy(rowidx_hbm.at[pl.ds(row0 + r, CHUNK)], src_v)
        pltpu.async_copy(acc_sh.at[src_v], rows_v, sem).wait()
        pltpu.sync_copy(rows_v, p_hbm.at[cid, pl.ds(row0 + r, CHUNK)])

    @pl.when(sid == NS - 1)
    def _out_tail():
        pltpu.sync_copy(rowidx_hbm.at[pl.ds(N - 16, 16)], idx16_v)
        pltpu.async_copy(acc_sh.at[idx16_v], rows_v.at[pl.ds(0, 16)],
                         sem).wait()
        pltpu.sync_copy(rows_v.at[pl.ds(0, 16)],
                        p_hbm.at[cid, pl.ds(N - 16, 16)])


_AGG_SCRATCH = (
    pltpu.VMEM((CHUNK,), jnp.int32),
    pltpu.VMEM((CHUNK,), jnp.int32),
    pltpu.VMEM((CHUNK, C), jnp.float32),
    pltpu.VMEM((16,), jnp.int32),
    pltpu.VMEM_SHARED((N, C), jnp.float32),
    pltpu.SemaphoreType.DMA,
)


@functools.partial(
    pl.kernel,
    out_type=jax.ShapeDtypeStruct((NC, N, C), jnp.float32),
    mesh=_mesh,
    scratch_types=list(_AGG_SCRATCH),
)
def _sc_agg(h_hbm, src_hbm, dst_hbm, rowidx_hbm, z_hbm, p_hbm,
            src_v, dst_v, rows_v, idx16_v, agg_sh, sem):
    cid = lax.axis_index("c")
    sid = lax.axis_index("s")
    wid = sid * NC + cid

    _zero_acc(agg_sh, rowidx_hbm, z_hbm, rows_v, src_v, idx16_v, sid)
    plsc.subcore_barrier()

    nck, base = _edge_range(wid)

    def chunk_body(j, carry):
        off = (base + j) * CHUNK
        pltpu.sync_copy(src_hbm.at[pl.ds(off, CHUNK)], src_v)
        pltpu.sync_copy(dst_hbm.at[pl.ds(off, CHUNK)], dst_v)
        pltpu.async_copy(h_hbm.at[src_v], rows_v, sem).wait()
        pltpu.sync_copy(rows_v, agg_sh.at[dst_v], add=True)
        return carry

    lax.fori_loop(0, nck, chunk_body, 0)
    plsc.subcore_barrier()

    _copy_out(agg_sh, rowidx_hbm, p_hbm, rows_v, src_v, idx16_v, sid, cid,
              sem)


@functools.partial(
    pl.kernel,
    out_type=jax.ShapeDtypeStruct((NC, N, C), jnp.float32),
    mesh=_mesh,
    scratch_types=list(_AGG_SCRATCH),
)
def _sc_cnt(dst_hbm, rowidx_hbm, z_hbm, ones_hbm, p_hbm,
            src_v, dst_v, rows_v, idx16_v, cnt_sh, sem):
    cid = lax.axis_index("c")
    sid = lax.axis_index("s")
    wid = sid * NC + cid

    _zero_acc(cnt_sh, rowidx_hbm, z_hbm, rows_v, src_v, idx16_v, sid)
    plsc.subcore_barrier()

    pltpu.sync_copy(ones_hbm, rows_v)
    nck, base = _edge_range(wid)

    def chunk_body(j, carry):
        off = (base + j) * CHUNK
        pltpu.sync_copy(dst_hbm.at[pl.ds(off, CHUNK)], dst_v)
        pltpu.sync_copy(rows_v, cnt_sh.at[dst_v], add=True)
        return carry

    lax.fori_loop(0, nck, chunk_body, 0)
    plsc.subcore_barrier()

    _copy_out(cnt_sh, rowidx_hbm, p_hbm, rows_v, src_v, idx16_v, sid, cid,
              sem)


_BLK = 1000


def _enc_body(x_ref, batch_ref, time_ref, seed_ref, er_ref, we_ref, wt_ref,
              be_ref, bt_ref, o_ref):
    lanes = lax.broadcasted_iota(jnp.int32, (1, B), 1)
    oh = jnp.where(batch_ref[...] == lanes, 1.0, 0.0)
    st = jnp.dot(oh, seed_ref[...], preferred_element_type=jnp.float32)
    rel = (st - time_ref[...].astype(jnp.float32)) / 86400.0
    j = lax.broadcasted_iota(jnp.int32, (1, C // 2), 1).astype(jnp.float32)
    w = jnp.exp(j * (-2.0 * jnp.log(10000.0) / C))
    ang = rel * w
    pe = jnp.concatenate([jnp.sin(ang), jnp.cos(ang)], axis=1)
    h = jnp.dot(x_ref[...], we_ref[...], preferred_element_type=jnp.float32)
    h = h + jnp.dot(pe, wt_ref[...], preferred_element_type=jnp.float32)
    o_ref[...] = h + be_ref[...] + bt_ref[...] + er_ref[...]


def _tc_encode(x, batch, time, seed_f, erows, W_enc, W_time, b_enc, b_time):
    return pl.pallas_call(
        _enc_body,
        grid=(N // _BLK,),
        in_specs=[
            pl.BlockSpec((_BLK, D_IN), lambda i: (i, 0)),
            pl.BlockSpec((_BLK, 1), lambda i: (i, 0)),
            pl.BlockSpec((_BLK, 1), lambda i: (i, 0)),
            pl.BlockSpec((B, 1), lambda i: (0, 0)),
            pl.BlockSpec((_BLK, C), lambda i: (i, 0)),
            pl.BlockSpec((D_IN, C), lambda i: (0, 0)),
            pl.BlockSpec((C, C), lambda i: (0, 0)),
            pl.BlockSpec((1, C), lambda i: (0, 0)),
            pl.BlockSpec((1, C), lambda i: (0, 0)),
        ],
        out_specs=pl.BlockSpec((_BLK, C), lambda i: (i, 0)),
        out_shape=jax.ShapeDtypeStruct((N, C), jnp.float32),
    )(x, batch, time, seed_f, erows, W_enc, W_time, b_enc, b_time)


def _comb_body(h_ref, p0_ref, p1_ref, c0_ref, c1_ref, ws_ref, wn_ref, b_ref,
               o_ref):
    cnt = c0_ref[...][:, :1] + c1_ref[...][:, :1]
    agg = (p0_ref[...] + p1_ref[...]) / jnp.maximum(cnt, 1.0)
    h = jnp.dot(h_ref[...], ws_ref[...], preferred_element_type=jnp.float32)
    h = h + jnp.dot(agg, wn_ref[...], preferred_element_type=jnp.float32)
    o_ref[...] = jnp.maximum(h + b_ref[...], 0.0)


def _tc_combine(h, p0, p1, c0, c1, Ws, Wn, b):
    return pl.pallas_call(
        _comb_body,
        grid=(N // _BLK,),
        in_specs=[
            pl.BlockSpec((_BLK, C), lambda i: (i, 0)),
            pl.BlockSpec((_BLK, C), lambda i: (i, 0)),
            pl.BlockSpec((_BLK, C), lambda i: (i, 0)),
            pl.BlockSpec((_BLK, C), lambda i: (i, 0)),
            pl.BlockSpec((_BLK, C), lambda i: (i, 0)),
            pl.BlockSpec((C, C), lambda i: (0, 0)),
            pl.BlockSpec((C, C), lambda i: (0, 0)),
            pl.BlockSpec((1, C), lambda i: (0, 0)),
        ],
        out_specs=pl.BlockSpec((_BLK, C), lambda i: (i, 0)),
        out_shape=jax.ShapeDtypeStruct((N, C), jnp.float32),
    )(h, p0, p1, c0, c1, Ws, Wn, b)


def _final_body(h_ref, q0_ref, q1_ref, c0_ref, c1_ref, ws_ref, wn_ref,
                b_ref, wh_ref, bh_ref, o_ref):
    cnt = c0_ref[...][:, :1] + c1_ref[...][:, :1]
    agg = (q0_ref[...] + q1_ref[...]) / jnp.maximum(cnt, 1.0)
    h = jnp.dot(h_ref[...], ws_ref[...], preferred_element_type=jnp.float32)
    h = h + jnp.dot(agg, wn_ref[...], preferred_element_type=jnp.float32)
    h = jnp.maximum(h + b_ref[...], 0.0)
    o_ref[...] = jnp.dot(h, wh_ref[...],
                         preferred_element_type=jnp.float32) + bh_ref[...]


def _tc_final(h, q0, q1, c0, c1, Ws, Wn, b, W_head, b_head):
    return pl.pallas_call(
        _final_body,
        grid=(1,),
        in_specs=[
            pl.BlockSpec((B, C), lambda i: (0, 0)),
            pl.BlockSpec((B, C), lambda i: (0, 0)),
            pl.BlockSpec((B, C), lambda i: (0, 0)),
            pl.BlockSpec((B, C), lambda i: (0, 0)),
            pl.BlockSpec((B, C), lambda i: (0, 0)),
            pl.BlockSpec((C, C), lambda i: (0, 0)),
            pl.BlockSpec((C, C), lambda i: (0, 0)),
            pl.BlockSpec((1, C), lambda i: (0, 0)),
            pl.BlockSpec((C, OUT), lambda i: (0, 0)),
            pl.BlockSpec((1, OUT), lambda i: (0, 0)),
        ],
        out_specs=pl.BlockSpec((B, OUT), lambda i: (0, 0)),
        out_shape=jax.ShapeDtypeStruct((B, OUT), jnp.float32),
    )(h, q0, q1, c0, c1, Ws, Wn, b, W_head, b_head)


def kernel(x, edge_index, seed_time, time, batch, n_id,
           W_enc, b_enc, W_time, b_time, emb,
           W_self0, W_neigh0, b0, W_self1, W_neigh1, b1,
           W_head, b_head):
    src = edge_index[0]
    dst = edge_index[1]
    rowidx = jnp.arange(N, dtype=jnp.int32)
    z128 = jnp.zeros((CHUNK, C), jnp.float32)
    ones128 = jnp.ones((CHUNK, C), jnp.float32)

    erows = _sc_prep(emb, n_id)
    cnt = _sc_cnt(dst, rowidx, z128, ones128)
    h0 = _tc_encode(x, batch.reshape(N, 1), time.reshape(N, 1),
                    seed_time.astype(jnp.float32).reshape(B, 1), erows,
                    W_enc, W_time, b_enc.reshape(1, C), b_time.reshape(1, C))

    p = _sc_agg(h0, src, dst, rowidx, z128)
    h1 = _tc_combine(h0, p[0], p[1], cnt[0], cnt[1], W_self0, W_neigh0,
                     b0.reshape(1, C))

    p2 = _sc_agg(h1, src, dst, rowidx, z128)
    out = _tc_final(h1, p2[0], p2[1], cnt[0], cnt[1], W_self1, W_neigh1,
                    b1.reshape(1, C), W_head, b_head.reshape(1, OUT))
    return out


# speedup vs baseline: 7.8494x; 1.4663x over previous
import functools

import jax
import jax.numpy as jnp
from jax import lax
from jax.experimental import pallas as pl
from jax.experimental.pallas import tpu as pltpu
from jax.experimental.pallas import tpu_sc as plsc

N = 10000
E = 320000
D_IN = 64
C = 128
B = 1024
V = 100000
OUT = 1

NC = 2
NS = 16
NW = NC * NS

CHUNK = 128
NCHUNKS = E // CHUNK
RPT = 624
NCHK_T = 80

_mesh = plsc.VectorSubcoreMesh(core_axis_name="c", subcore_axis_name="s",
                               num_cores=NC, num_subcores=NS)


@functools.partial(
    pl.kernel,
    out_type=jax.ShapeDtypeStruct((N, C), jnp.float32),
    mesh=_mesh,
    scratch_types=[
        pltpu.VMEM((CHUNK,), jnp.int32),
        pltpu.VMEM((CHUNK, C), jnp.float32),
        pltpu.SemaphoreType.DMA,
    ],
)
def _sc_prep(emb_hbm, nid_hbm, erows_hbm, nid_v, erow_v, sem):
    wid = lax.axis_index("s") * NC + lax.axis_index("c")
    nck = jnp.where(wid < 14, 3, 2)
    base = jnp.where(wid < 14, 3 * wid, 2 * wid + 14)

    def chunk_body(j, carry):
        off = (base + j) * CHUNK
        pltpu.sync_copy(nid_hbm.at[pl.ds(off, CHUNK)], nid_v)
        pltpu.async_copy(emb_hbm.at[nid_v], erow_v, sem).wait()
        pltpu.sync_copy(erow_v, erows_hbm.at[pl.ds(off, CHUNK)])
        return carry

    lax.fori_loop(0, nck, chunk_body, 0)

    @pl.when(wid == NW - 1)
    def _tail():
        toff = N - 16
        pltpu.sync_copy(nid_hbm.at[pl.ds(toff, 16)], nid_v.at[pl.ds(0, 16)])
        pltpu.async_copy(emb_hbm.at[nid_v.at[pl.ds(0, 16)]],
                         erow_v.at[pl.ds(0, 16)], sem).wait()
        pltpu.sync_copy(erow_v.at[pl.ds(0, 16)],
                        erows_hbm.at[pl.ds(toff, 16)])


_PIECES = (0, 128, 256, 384, 496)


def _edge_range(wid):
    nck = jnp.where(wid < 4, 79, 78)
    base = wid * 78 + jnp.minimum(wid, 4)
    return nck, base


def _zero_acc(acc_sh, rowidx_hbm, z_hbm, rows_v, src_v, idx16_v, sid):
    pltpu.sync_copy(z_hbm, rows_v)
    row0 = sid * RPT
    for r in _PIECES:
        pltpu.sync_copy(rowidx_hbm.at[pl.ds(row0 + r, CHUNK)], src_v)
        pltpu.sync_copy(rows_v, acc_sh.at[src_v])

    @pl.when(sid == NS - 1)
    def _zero_tail():
        pltpu.sync_copy(rowidx_hbm.at[pl.ds(N - 16, 16)], idx16_v)
        pltpu.sync_copy(rows_v.at[pl.ds(0, 16)], acc_sh.at[idx16_v])


def _copy_out(acc_sh, rowidx_hbm, p_hbm, rows_v, src_v, idx16_v, sid, cid,
              sem):
    row0 = sid * RPT
    for r in _PIECES:
        pltpu.sync_copy(rowidx_hbm.at[pl.ds(row0 + r, CHUNK)], src_v)
        pltpu.async_copy(acc_sh.at[src_v], rows_v, sem).wait()
        pltpu.sync_copy(rows_v, p_hbm.at[cid, pl.ds(row0 + r, CHUNK)])

    @pl.when(sid == NS - 1)
    def _out_tail():
        pltpu.sync_copy(rowidx_hbm.at[pl.ds(N - 16, 16)], idx16_v)
        pltpu.async_copy(acc_sh.at[idx16_v], rows_v.at[pl.ds(0, 16)],
                         sem).wait()
        pltpu.sync_copy(rows_v.at[pl.ds(0, 16)],
                        p_hbm.at[cid, pl.ds(N - 16, 16)])


@functools.partial(
    pl.kernel,
    out_type=jax.ShapeDtypeStruct((NC, N, C), jnp.float32),
    mesh=_mesh,
    scratch_types=[
        pltpu.VMEM((CHUNK,), jnp.int32),
        pltpu.VMEM((CHUNK,), jnp.int32),
        pltpu.VMEM((CHUNK,), jnp.int32),
        pltpu.VMEM((CHUNK,), jnp.int32),
        pltpu.VMEM((CHUNK, C), jnp.float32),
        pltpu.VMEM((CHUNK, C), jnp.float32),
        pltpu.VMEM((16,), jnp.int32),
        pltpu.VMEM_SHARED((N, C), jnp.float32),
        pltpu.SemaphoreType.DMA,
        pltpu.SemaphoreType.DMA,
    ],
)
def _sc_agg(h_hbm, src_hbm, dst_hbm, rowidx_hbm, z_hbm, p_hbm,
            sv0, sv1, dv0, dv1, rows0, rows1, idx16_v, agg_sh, sem0, sem1):
    cid = lax.axis_index("c")
    sid = lax.axis_index("s")
    wid = sid * NC + cid

    _zero_acc(agg_sh, rowidx_hbm, z_hbm, rows0, sv0, idx16_v, sid)
    plsc.subcore_barrier()

    base = wid * 78 + jnp.minimum(wid, 4)
    NPAIR = 39

    def _stage_idx(c, sv, dv):
        off = (base + c) * CHUNK
        pltpu.sync_copy(src_hbm.at[pl.ds(off, CHUNK)], sv)
        pltpu.sync_copy(dst_hbm.at[pl.ds(off, CHUNK)], dv)

    _stage_idx(0, sv0, dv0)
    gd0 = pltpu.async_copy(h_hbm.at[sv0], rows0, sem0)

    def pair_body(j2, carry):
        c0 = 2 * j2
        _stage_idx(c0 + 1, sv1, dv1)
        pltpu.async_copy(h_hbm.at[sv1], rows1, sem1)
        pltpu.make_async_copy(h_hbm.at[sv0], rows0, sem0).wait()
        pltpu.sync_copy(rows0, agg_sh.at[dv0], add=True)

        @pl.when(j2 < NPAIR - 1)
        def _prefetch():
            _stage_idx(c0 + 2, sv0, dv0)
            pltpu.async_copy(h_hbm.at[sv0], rows0, sem0)

        pltpu.make_async_copy(h_hbm.at[sv1], rows1, sem1).wait()
        pltpu.sync_copy(rows1, agg_sh.at[dv1], add=True)
        return carry

    lax.fori_loop(0, NPAIR, pair_body, 0)

    @pl.when(wid < 4)
    def _extra_chunk():
        _stage_idx(78, sv0, dv0)
        pltpu.async_copy(h_hbm.at[sv0], rows0, sem0).wait()
        pltpu.sync_copy(rows0, agg_sh.at[dv0], add=True)

    plsc.subcore_barrier()
    _copy_out(agg_sh, rowidx_hbm, p_hbm, rows0, sv0, idx16_v, sid, cid,
              sem0)


@functools.partial(
    pl.kernel,
    out_type=jax.ShapeDtypeStruct((NC, N, C), jnp.float32),
    mesh=_mesh,
    scratch_types=[
        pltpu.VMEM((CHUNK,), jnp.int32),
        pltpu.VMEM((NCHK_T, 1, CHUNK), jnp.int32),
        pltpu.VMEM((CHUNK, C), jnp.float32),
        pltpu.VMEM((16,), jnp.int32),
        pltpu.VMEM_SHARED((N, C), jnp.float32),
        pltpu.SemaphoreType.DMA,
    ],
)
def _sc_cnt(dst3_hbm, rowidx_hbm, z_hbm, ones_hbm, p_hbm,
            src_v, dstbuf, rows_v, idx16_v, cnt_sh, sem):
    cid = lax.axis_index("c")
    sid = lax.axis_index("s")
    wid = sid * NC + cid

    _zero_acc(cnt_sh, rowidx_hbm, z_hbm, rows_v, src_v, idx16_v, sid)
    plsc.subcore_barrier()

    nck = jnp.where(wid < 4, 79, 78)
    base = wid * 78 + jnp.minimum(wid, 4)
    pltpu.sync_copy(dst3_hbm.at[pl.ds(base, NCHK_T)], dstbuf)
    pltpu.sync_copy(ones_hbm, rows_v)

    def fire(j, carry):
        pltpu.async_copy(rows_v, cnt_sh.at[dstbuf.at[j, 0]], sem, add=True)
        return carry

    lax.fori_loop(0, nck, fire, 0)

    def drain(j, carry):
        pltpu.make_async_copy(rows_v, cnt_sh.at[dstbuf.at[j, 0]], sem).wait()
        return carry

    lax.fori_loop(0, nck, drain, 0)
    plsc.subcore_barrier()

    _copy_out(cnt_sh, rowidx_hbm, p_hbm, rows_v, src_v, idx16_v, sid, cid,
              sem)


_BLK = 1000


def _enc_body(x_ref, batch_ref, time_ref, seed_ref, er_ref, we_ref, wt_ref,
              be_ref, bt_ref, o_ref):
    lanes = lax.broadcasted_iota(jnp.int32, (1, B), 1)
    oh = jnp.where(batch_ref[...] == lanes, 1.0, 0.0)
    st = jnp.dot(oh, seed_ref[...], preferred_element_type=jnp.float32)
    rel = (st - time_ref[...].astype(jnp.float32)) / 86400.0
    j = lax.broadcasted_iota(jnp.int32, (1, C // 2), 1).astype(jnp.float32)
    w = jnp.exp(j * (-2.0 * jnp.log(10000.0) / C))
    ang = rel * w
    pe = jnp.concatenate([jnp.sin(ang), jnp.cos(ang)], axis=1)
    h = jnp.dot(x_ref[...], we_ref[...], preferred_element_type=jnp.float32)
    h = h + jnp.dot(pe, wt_ref[...], preferred_element_type=jnp.float32)
    o_ref[...] = h + be_ref[...] + bt_ref[...] + er_ref[...]


def _tc_encode(x, batch, time, seed_f, erows, W_enc, W_time, b_enc, b_time):
    return pl.pallas_call(
        _enc_body,
        grid=(N // _BLK,),
        in_specs=[
            pl.BlockSpec((_BLK, D_IN), lambda i: (i, 0)),
            pl.BlockSpec((_BLK, 1), lambda i: (i, 0)),
            pl.BlockSpec((_BLK, 1), lambda i: (i, 0)),
            pl.BlockSpec((B, 1), lambda i: (0, 0)),
            pl.BlockSpec((_BLK, C), lambda i: (i, 0)),
            pl.BlockSpec((D_IN, C), lambda i: (0, 0)),
            pl.BlockSpec((C, C), lambda i: (0, 0)),
            pl.BlockSpec((1, C), lambda i: (0, 0)),
            pl.BlockSpec((1, C), lambda i: (0, 0)),
        ],
        out_specs=pl.BlockSpec((_BLK, C), lambda i: (i, 0)),
        out_shape=jax.ShapeDtypeStruct((N, C), jnp.float32),
    )(x, batch, time, seed_f, erows, W_enc, W_time, b_enc, b_time)


def _comb_body(h_ref, p0_ref, p1_ref, c0_ref, c1_ref, ws_ref, wn_ref, b_ref,
               o_ref):
    cnt = c0_ref[...][:, :1] + c1_ref[...][:, :1]
    agg = (p0_ref[...] + p1_ref[...]) / jnp.maximum(cnt, 1.0)
    h = jnp.dot(h_ref[...], ws_ref[...], preferred_element_type=jnp.float32)
    h = h + jnp.dot(agg, wn_ref[...], preferred_element_type=jnp.float32)
    o_ref[...] = jnp.maximum(h + b_ref[...], 0.0)


def _tc_combine(h, p0, p1, c0, c1, Ws, Wn, b):
    return pl.pallas_call(
        _comb_body,
        grid=(N // _BLK,),
        in_specs=[
            pl.BlockSpec((_BLK, C), lambda i: (i, 0)),
            pl.BlockSpec((_BLK, C), lambda i: (i, 0)),
            pl.BlockSpec((_BLK, C), lambda i: (i, 0)),
            pl.BlockSpec((_BLK, C), lambda i: (i, 0)),
            pl.BlockSpec((_BLK, C), lambda i: (i, 0)),
            pl.BlockSpec((C, C), lambda i: (0, 0)),
            pl.BlockSpec((C, C), lambda i: (0, 0)),
            pl.BlockSpec((1, C), lambda i: (0, 0)),
        ],
        out_specs=pl.BlockSpec((_BLK, C), lambda i: (i, 0)),
        out_shape=jax.ShapeDtypeStruct((N, C), jnp.float32),
    )(h, p0, p1, c0, c1, Ws, Wn, b)


def _final_body(h_ref, q0_ref, q1_ref, c0_ref, c1_ref, ws_ref, wn_ref,
                b_ref, wh_ref, bh_ref, o_ref):
    cnt = c0_ref[...][:, :1] + c1_ref[...][:, :1]
    agg = (q0_ref[...] + q1_ref[...]) / jnp.maximum(cnt, 1.0)
    h = jnp.dot(h_ref[...], ws_ref[...], preferred_element_type=jnp.float32)
    h = h + jnp.dot(agg, wn_ref[...], preferred_element_type=jnp.float32)
    h = jnp.maximum(h + b_ref[...], 0.0)
    o_ref[...] = jnp.dot(h, wh_ref[...],
                         preferred_element_type=jnp.float32) + bh_ref[...]


def _tc_final(h, q0, q1, c0, c1, Ws, Wn, b, W_head, b_head):
    return pl.pallas_call(
        _final_body,
        grid=(1,),
        in_specs=[
            pl.BlockSpec((B, C), lambda i: (0, 0)),
            pl.BlockSpec((B, C), lambda i: (0, 0)),
            pl.BlockSpec((B, C), lambda i: (0, 0)),
            pl.BlockSpec((B, C), lambda i: (0, 0)),
            pl.BlockSpec((B, C), lambda i: (0, 0)),
            pl.BlockSpec((C, C), lambda i: (0, 0)),
            pl.BlockSpec((C, C), lambda i: (0, 0)),
            pl.BlockSpec((1, C), lambda i: (0, 0)),
            pl.BlockSpec((C, OUT), lambda i: (0, 0)),
            pl.BlockSpec((1, OUT), lambda i: (0, 0)),
        ],
        out_specs=pl.BlockSpec((B, OUT), lambda i: (0, 0)),
        out_shape=jax.ShapeDtypeStruct((B, OUT), jnp.float32),
    )(h, q0, q1, c0, c1, Ws, Wn, b, W_head, b_head)


def kernel(x, edge_index, seed_time, time, batch, n_id,
           W_enc, b_enc, W_time, b_time, emb,
           W_self0, W_neigh0, b0, W_self1, W_neigh1, b1,
           W_head, b_head):
    src = edge_index[0]
    dst = edge_index[1]
    dst3 = jnp.pad(dst, (0, 2560 * CHUNK - E)).reshape(2560, 1, CHUNK)
    rowidx = jnp.arange(N, dtype=jnp.int32)
    z128 = jnp.zeros((CHUNK, C), jnp.float32)
    ones128 = jnp.ones((CHUNK, C), jnp.float32)

    erows = _sc_prep(emb, n_id)
    cnt = _sc_cnt(dst3, rowidx, z128, ones128)
    h0 = _tc_encode(x, batch.reshape(N, 1), time.reshape(N, 1),
                    seed_time.astype(jnp.float32).reshape(B, 1), erows,
                    W_enc, W_time, b_enc.reshape(1, C), b_time.reshape(1, C))

    p = _sc_agg(h0, src, dst, rowidx, z128)
    h1 = _tc_combine(h0, p[0], p[1], cnt[0], cnt[1], W_self0, W_neigh0,
                     b0.reshape(1, C))

    p2 = _sc_agg(h1, src, dst, rowidx, z128)
    out = _tc_final(h1, p2[0], p2[1], cnt[0], cnt[1], W_self1, W_neigh1,
                    b1.reshape(1, C), W_head, b_head.reshape(1, OUT))
    return out
